# baseline (speedup 1.0000x reference)
import functools

import jax
import jax.numpy as jnp
from jax import lax
from jax.experimental import pallas as pl
from jax.experimental.pallas import tpu as pltpu
from jax.experimental.pallas import tpu_sc as plsc

_LANES = 16
_SC_ROWS = 2048
_TC_BS = 512


def _make_sc_tail_add(B, S, D, R):
    NC, NS = 2, 16
    NW = NC * NS
    rows_per_w = R // NW
    CH = 16
    n_chunks = rows_per_w // CH
    chunk = CH * D
    x_base = (B * S - R) * D
    pe_base = (S - R) * D

    mesh = plsc.VectorSubcoreMesh(core_axis_name="c", subcore_axis_name="s")

    @functools.partial(
        pl.kernel,
        out_type=jax.ShapeDtypeStruct((R * D,), jnp.float32),
        mesh=mesh,
        scratch_types=[
            [pltpu.VMEM((chunk,), jnp.float32) for _ in range(2)],
            [pltpu.VMEM((chunk,), jnp.float32) for _ in range(2)],
            [pltpu.VMEM((chunk,), jnp.float32) for _ in range(2)],
            [pltpu.SemaphoreType.DMA for _ in range(2)],
            [pltpu.SemaphoreType.DMA for _ in range(2)],
            [pltpu.SemaphoreType.DMA for _ in range(2)],
        ],
    )
    def sc_add(x_hbm, pe_hbm, out_hbm, x_v, o_v, pe_v, sx, so, sp):
        wid = lax.axis_index("s") * NC + lax.axis_index("c")
        base = wid * rows_per_w * D

        def off(k):
            return base + k * chunk

        for j in range(2):
            pltpu.async_copy(
                x_hbm.at[pl.ds(x_base + off(j), chunk)], x_v[j], sx[j]
            )
            pltpu.async_copy(
                pe_hbm.at[pl.ds(pe_base + off(j), chunk)], pe_v[j], sp[j]
            )

        def q_body(q, carry):
            for jj in range(2):
                k = 2 * q + jj
                @pl.when(k >= 2)
                def _():
                    pltpu.make_async_copy(
                        o_v[jj], out_hbm.at[pl.ds(off(k - 2), chunk)], so[jj]
                    ).wait()
                pltpu.make_async_copy(
                    x_hbm.at[pl.ds(x_base + off(k), chunk)], x_v[jj], sx[jj]
                ).wait()
                pltpu.make_async_copy(
                    pe_hbm.at[pl.ds(pe_base + off(k), chunk)], pe_v[jj],
                    sp[jj],
                ).wait()

                @plsc.parallel_loop(0, chunk // _LANES, unroll=8)
                def _add(i):
                    sl = pl.ds(i * _LANES, _LANES)
                    o_v[jj][sl] = x_v[jj][sl] + pe_v[jj][sl]

                pltpu.async_copy(
                    o_v[jj], out_hbm.at[pl.ds(off(k), chunk)], so[jj]
                )

                @pl.when(k + 2 < n_chunks)
                def _():
                    pltpu.async_copy(
                        x_hbm.at[pl.ds(x_base + off(k + 2), chunk)], x_v[jj],
                        sx[jj],
                    )
                    pltpu.async_copy(
                        pe_hbm.at[pl.ds(pe_base + off(k + 2), chunk)],
                        pe_v[jj], sp[jj],
                    )
            return carry

        lax.fori_loop(0, n_chunks // 2, q_body, 0)

        for k in (n_chunks - 2, n_chunks - 1):
            pltpu.make_async_copy(
                o_v[k % 2], out_hbm.at[pl.ds(off(k), chunk)], so[k % 2]
            ).wait()

    return sc_add


def _add_body(x_ref, pe_ref, o_ref):
    o_ref[...] = x_ref[...] + pe_ref[...]


def _tc_head_add(x2, pe, rows_tc, S):
    D = x2.shape[1]
    n_pe_blocks = S // _TC_BS
    return pl.pallas_call(
        _add_body,
        grid=(rows_tc // _TC_BS,),
        in_specs=[
            pl.BlockSpec((_TC_BS, D), lambda i: (i, 0)),
            pl.BlockSpec((_TC_BS, D), lambda i: (i % n_pe_blocks, 0)),
        ],
        out_specs=pl.BlockSpec((_TC_BS, D), lambda i: (i, 0)),
        out_shape=jax.ShapeDtypeStruct((rows_tc, D), x2.dtype),
    )(x2, pe)


def kernel(x, pos_embedding):
    B, S, D = x.shape
    R = _SC_ROWS
    rows_tc = B * S - R
    x2 = x.reshape(B * S, D)
    tc_out = _tc_head_add(x2, pos_embedding, rows_tc, S)
    sc_out = _make_sc_tail_add(B, S, D, R)(
        x.reshape(-1), pos_embedding.reshape(-1)
    )
    out = jnp.concatenate([tc_out.reshape(-1), sc_out])
    return out.reshape(B, S, D)


# speedup vs baseline: 5.9506x; 5.9506x over previous
import jax
import jax.numpy as jnp
from jax.experimental import pallas as pl


def _add_body(x_ref, pe_ref, o_ref):
    o_ref[...] = x_ref[...] + pe_ref[...]


def kernel(x, pos_embedding):
    B, S, D = x.shape
    BS = 1024
    return pl.pallas_call(
        _add_body,
        grid=(S // BS, B),
        in_specs=[
            pl.BlockSpec((1, BS, D), lambda s, b: (b, s, 0)),
            pl.BlockSpec((BS, D), lambda s, b: (s, 0)),
        ],
        out_specs=pl.BlockSpec((1, BS, D), lambda s, b: (b, s, 0)),
        out_shape=jax.ShapeDtypeStruct((B, S, D), x.dtype),
    )(x, pos_embedding)
